# baseline (speedup 1.0000x reference)
import jax
import jax.numpy as jnp
from jax import lax
from jax.experimental import pallas as pl

_B = 8
_MAX_LEN = 2048


def _body(p_ref, l_ref, out_ref, rs_ref):
    l = l_ref[...]
    col = lax.broadcasted_iota(jnp.int32, (_B, _MAX_LEN), 1)
    out_ref[...] = jnp.where(col < l, p_ref[...], 0)
    j = lax.broadcasted_iota(jnp.int32, (_B, 16), 0)
    k = lax.broadcasted_iota(jnp.int32, (_B, 16), 1)
    mat = jnp.where(j < k, jnp.broadcast_to(l, (_B, 16)), 0)
    rs_ref[...] = jnp.sum(mat, axis=0, keepdims=True)


def kernel(pieces, lengths):
    out, rs = pl.pallas_call(
        _body,
        out_shape=[
            jax.ShapeDtypeStruct((_B, _MAX_LEN), jnp.int32),
            jax.ShapeDtypeStruct((1, 16), jnp.int32),
        ],
    )(pieces, lengths.reshape(_B, 1))
    return out, rs[0, : _B + 1]
